# Initial kernel scaffold; baseline (speedup 1.0000x reference)
import jax
import jax.numpy as jnp
from jax.experimental import pallas as pl


def kernel(msg, x_i, x_j, e_ij, index, num_nodes, W, b):
    raise NotImplementedError("write your pallas kernel here")


# speedup vs baseline: 1.7563x; 1.7563x over previous
import functools

import jax
import jax.numpy as jnp
from jax import lax
from jax.experimental import pallas as pl
from jax.experimental.pallas import tpu as pltpu
from jax.experimental.pallas import tpu_sc as plsc

E = 320000
NN = 10000
D = 128
DE = 16

NC = 2
NS = 16
NW = NC * NS
EPW = E // NW
CH = 80
NCHUNK = EPW // CH
RPT = 624
RPT_LAST = 640
ZR = 80

GATE_BLK = 8192
ADD_BLK = 2000


def _gate_body(xj_ref, ei_ref, xi_ref, w1_ref, w2_ref, w3_ref, b_ref, out_ref):
    s = jnp.sum(xj_ref[...] * w1_ref[...][None, :], axis=1)
    s += jnp.sum(ei_ref[...] * w2_ref[...][None, :], axis=1)
    s += jnp.sum(xi_ref[...] * w3_ref[...][None, :], axis=1)
    out_ref[...] = jnp.tanh(s + b_ref[0])


_gate_call = pl.pallas_call(
    _gate_body,
    grid=(pl.cdiv(E, GATE_BLK),),
    in_specs=[
        pl.BlockSpec((GATE_BLK, D), lambda i: (i, 0)),
        pl.BlockSpec((GATE_BLK, DE), lambda i: (i, 0)),
        pl.BlockSpec((GATE_BLK, D), lambda i: (i, 0)),
        pl.BlockSpec((D,), lambda i: (0,)),
        pl.BlockSpec((DE,), lambda i: (0,)),
        pl.BlockSpec((D,), lambda i: (0,)),
        pl.BlockSpec((1,), lambda i: (0,)),
    ],
    out_specs=pl.BlockSpec((GATE_BLK,), lambda i: (i,)),
    out_shape=jax.ShapeDtypeStruct((E,), jnp.float32),
)


_mesh = plsc.VectorSubcoreMesh(core_axis_name="c", subcore_axis_name="s")


@functools.partial(
    pl.kernel,
    mesh=_mesh,
    out_type=jax.ShapeDtypeStruct((NC * NN, D), jnp.float32),
    scratch_types=[
        pltpu.VMEM((CH, D), jnp.float32),
        pltpu.VMEM((CH,), jnp.float32),
        pltpu.VMEM((8, CH), jnp.int32),
        pltpu.VMEM((ZR, D), jnp.float32),
        pltpu.VMEM_SHARED((NN, D), jnp.float32),
    ],
    compiler_params=pltpu.CompilerParams(needs_layout_passes=False),
)
def _sc_scatter(msg_hbm, w_hbm, idx_hbm, out_hbm, msg_v, w_v, idx_v, z_v, acc):
    cid = lax.axis_index("c")
    sid = lax.axis_index("s")
    wid = cid * NS + sid
    base = wid * EPW

    def _zrow(r, carry):
        for c in range(D // 16):
            z_v[r, pl.ds(c * 16, 16)] = jnp.zeros((16,), jnp.float32)
        return carry

    lax.fori_loop(0, ZR, _zrow, 0)
    for k in range(RPT_LAST // ZR):
        pltpu.sync_copy(z_v, acc.at[pl.ds(sid * RPT + k * ZR, ZR)])
    plsc.subcore_barrier()

    def _chunk(j, carry):
        cb = base + j * CH
        pltpu.sync_copy(msg_hbm.at[pl.ds(cb, CH)], msg_v)
        pltpu.sync_copy(w_hbm.at[pl.ds(cb, CH)], w_v)
        pltpu.sync_copy(idx_hbm.at[pl.ds(cb, CH)], idx_v.at[0])

        def _mrow(r, inner):
            wb = plsc.load_gather(w_v, [jnp.full((16,), r, jnp.int32)])
            for c in range(D // 16):
                sl = pl.ds(c * 16, 16)
                msg_v[r, sl] = msg_v[r, sl] * wb
            return inner

        lax.fori_loop(0, CH, _mrow, 0)
        pltpu.sync_copy(msg_v, acc.at[idx_v.at[0]], add=True)
        return carry

    lax.fori_loop(0, NCHUNK, _chunk, 0)
    plsc.subcore_barrier()

    @pl.when(sid < NS - 1)
    def _drain_body():
        pltpu.sync_copy(
            acc.at[pl.ds(sid * RPT, RPT)],
            out_hbm.at[pl.ds(cid * NN + sid * RPT, RPT)],
        )

    @pl.when(sid == NS - 1)
    def _drain_last():
        pltpu.sync_copy(
            acc.at[pl.ds((NS - 1) * RPT, RPT_LAST)],
            out_hbm.at[pl.ds(cid * NN + (NS - 1) * RPT, RPT_LAST)],
        )


def _add_body(a_ref, b_ref, o_ref):
    o_ref[...] = a_ref[...] + b_ref[...]


_combine_call = pl.pallas_call(
    _add_body,
    grid=(NN // ADD_BLK,),
    in_specs=[
        pl.BlockSpec((ADD_BLK, D), lambda i: (i, 0)),
        pl.BlockSpec((ADD_BLK, D), lambda i: (i, 0)),
    ],
    out_specs=pl.BlockSpec((ADD_BLK, D), lambda i: (i, 0)),
    out_shape=jax.ShapeDtypeStruct((NN, D), jnp.float32),
)


def kernel(msg, x_i, x_j, e_ij, index, num_nodes, W, b):
    w1 = W[:D, 0]
    w2 = W[D:D + DE, 0]
    w3 = W[D + DE:, 0]
    gate = _gate_call(x_j, e_ij, x_i, w1, w2, w3, b)
    idx = jnp.minimum(index, num_nodes - 1).astype(jnp.int32)
    parts = _sc_scatter(msg, gate, idx)
    return _combine_call(parts[:NN], parts[NN:])


# speedup vs baseline: 2.8382x; 1.6160x over previous
import functools

import jax
import jax.numpy as jnp
from jax import lax
from jax.experimental import pallas as pl
from jax.experimental.pallas import tpu as pltpu
from jax.experimental.pallas import tpu_sc as plsc

E = 320000
NN = 10000
D = 128
DE = 16

NC = 2
NS = 16
NW = NC * NS
EPW = E // NW
CH = 80
NCHUNK = EPW // CH
RPT = 624
RPT_LAST = 640
ZR = 80

GATE_BLK = 8192
ADD_BLK = 2000


def _gate_body(xj_ref, ei_ref, xi_ref, w1_ref, w2_ref, w3_ref, b_ref, out_ref):
    dn = (((1,), (0,)), ((), ()))
    s = jax.lax.dot_general(xj_ref[...], w1_ref[...], dn,
                            preferred_element_type=jnp.float32)
    s = s + jax.lax.dot_general(ei_ref[...], w2_ref[...], dn,
                                preferred_element_type=jnp.float32)
    s = s + jax.lax.dot_general(xi_ref[...], w3_ref[...], dn,
                                preferred_element_type=jnp.float32)
    out_ref[...] = jnp.tanh(s[:, 0] + b_ref[0])


_gate_call = pl.pallas_call(
    _gate_body,
    grid=(pl.cdiv(E, GATE_BLK),),
    in_specs=[
        pl.BlockSpec((GATE_BLK, D), lambda i: (i, 0)),
        pl.BlockSpec((GATE_BLK, DE), lambda i: (i, 0)),
        pl.BlockSpec((GATE_BLK, D), lambda i: (i, 0)),
        pl.BlockSpec((D, 1), lambda i: (0, 0)),
        pl.BlockSpec((DE, 1), lambda i: (0, 0)),
        pl.BlockSpec((D, 1), lambda i: (0, 0)),
        pl.BlockSpec((1,), lambda i: (0,)),
    ],
    out_specs=pl.BlockSpec((GATE_BLK,), lambda i: (i,)),
    out_shape=jax.ShapeDtypeStruct((E,), jnp.float32),
)


_mesh = plsc.VectorSubcoreMesh(core_axis_name="c", subcore_axis_name="s")


@functools.partial(
    pl.kernel,
    mesh=_mesh,
    out_type=jax.ShapeDtypeStruct((NC * NN, D), jnp.float32),
    scratch_types=[
        pltpu.VMEM((2, CH, D), jnp.float32),
        pltpu.VMEM((2, CH), jnp.float32),
        pltpu.VMEM((2, CH), jnp.int32),
        pltpu.VMEM((ZR, D), jnp.float32),
        pltpu.VMEM_SHARED((NN, D), jnp.float32),
        pltpu.SemaphoreType.DMA,
        pltpu.SemaphoreType.DMA,
    ],
    compiler_params=pltpu.CompilerParams(needs_layout_passes=False),
)
def _sc_scatter(msg_hbm, w_hbm, idx_hbm, out_hbm, msg_v, w_v, idx_v, z_v, acc,
                sem0, sem1):
    cid = lax.axis_index("c")
    sid = lax.axis_index("s")
    wid = cid * NS + sid
    base = wid * EPW
    sems = (sem0, sem1)

    def _zrow(r, carry):
        for c in range(D // 16):
            z_v[r, pl.ds(c * 16, 16)] = jnp.zeros((16,), jnp.float32)
        return carry

    lax.fori_loop(0, ZR, _zrow, 0)
    for k in range(RPT_LAST // ZR):
        pltpu.sync_copy(z_v, acc.at[pl.ds(sid * RPT + k * ZR, ZR)])
    plsc.subcore_barrier()

    def _dmas(j, b):
        cb = base + j * CH
        return (
            pltpu.make_async_copy(msg_hbm.at[pl.ds(cb, CH)], msg_v.at[b],
                                  sems[b]),
            pltpu.make_async_copy(w_hbm.at[pl.ds(cb, CH)], w_v.at[b], sems[b]),
            pltpu.make_async_copy(idx_hbm.at[pl.ds(cb, CH)], idx_v.at[b],
                                  sems[b]),
        )

    def _start(j, b):
        for d in _dmas(j, b):
            d.start()

    def _process(j, b):
        for d in _dmas(j, b):
            d.wait()

        def _mrow(r, inner):
            wb = plsc.load_gather(w_v.at[b], [jnp.full((16,), r, jnp.int32)])
            for c in range(D // 16):
                sl = pl.ds(c * 16, 16)
                msg_v[b, r, sl] = msg_v[b, r, sl] * wb
            return inner

        lax.fori_loop(0, CH, _mrow, 0)
        pltpu.sync_copy(msg_v.at[b], acc.at[idx_v.at[b]], add=True)

    _start(0, 0)

    def _pair(k, carry):
        j0 = 2 * k
        _start(j0 + 1, 1)
        _process(j0, 0)
        _start(j0 + 2, 0)
        _process(j0 + 1, 1)
        return carry

    lax.fori_loop(0, (NCHUNK - 1) // 2, _pair, 0)
    _process(NCHUNK - 1, 0)
    plsc.subcore_barrier()

    @pl.when(sid < NS - 1)
    def _drain_body():
        pltpu.sync_copy(
            acc.at[pl.ds(sid * RPT, RPT)],
            out_hbm.at[pl.ds(cid * NN + sid * RPT, RPT)],
        )

    @pl.when(sid == NS - 1)
    def _drain_last():
        pltpu.sync_copy(
            acc.at[pl.ds((NS - 1) * RPT, RPT_LAST)],
            out_hbm.at[pl.ds(cid * NN + (NS - 1) * RPT, RPT_LAST)],
        )


def _add_body(a_ref, b_ref, o_ref):
    o_ref[...] = a_ref[...] + b_ref[...]


_combine_call = pl.pallas_call(
    _add_body,
    grid=(NN // ADD_BLK,),
    in_specs=[
        pl.BlockSpec((ADD_BLK, D), lambda i: (i, 0)),
        pl.BlockSpec((ADD_BLK, D), lambda i: (i, 0)),
    ],
    out_specs=pl.BlockSpec((ADD_BLK, D), lambda i: (i, 0)),
    out_shape=jax.ShapeDtypeStruct((NN, D), jnp.float32),
)


def kernel(msg, x_i, x_j, e_ij, index, num_nodes, W, b):
    w1 = W[:D]
    w2 = W[D:D + DE]
    w3 = W[D + DE:]
    gate = _gate_call(x_j, e_ij, x_i, w1, w2, w3, b)
    idx = jnp.minimum(index, num_nodes - 1).astype(jnp.int32)
    parts = _sc_scatter(msg, gate, idx)
    return _combine_call(parts[:NN], parts[NN:])
